# baseline (speedup 1.0000x reference)
import functools

import jax
import jax.numpy as jnp
from jax import lax
from jax.experimental import pallas as pl
from jax.experimental.pallas import tpu as pltpu
from jax.experimental.pallas import tpu_sc as plsc

_D = 1024
_LANES = 16
_CH = 16


def _sc_add(x2, t2, B, S):
    R = B * S
    info = plsc.get_sparse_core_info()
    NC, NS = info.num_cores, info.num_subcores
    NW = NC * NS
    rows_per_w = R // NW
    nch = rows_per_w // _CH
    chunk_elems = _CH * _D

    mesh = plsc.VectorSubcoreMesh(core_axis_name="c", subcore_axis_name="s")

    @functools.partial(
        pl.kernel,
        mesh=mesh,
        out_type=jax.ShapeDtypeStruct((R * _D,), jnp.float32),
        scratch_types=[
            pltpu.VMEM((chunk_elems,), jnp.float32),
            pltpu.VMEM((chunk_elems,), jnp.float32),
            pltpu.SemaphoreType.DMA,
            pltpu.SemaphoreType.DMA,
        ],
    )
    def k(x_hbm, t_hbm, out_hbm, o_v, t_v, sem_x, sem_t):
        wid = lax.axis_index("s") * NC + lax.axis_index("c")
        row0 = wid * rows_per_w
        s0 = row0 % S

        def chunk_body(i, _):
            xoff = (row0 + i * _CH) * _D
            toff = (s0 + i * _CH) * _D
            cx = pltpu.async_copy(x_hbm.at[pl.ds(xoff, chunk_elems)], o_v, sem_x)
            ct = pltpu.async_copy(t_hbm.at[pl.ds(toff, chunk_elems)], t_v, sem_t)
            cx.wait()
            ct.wait()

            def add_body(j, _):
                sl = pl.ds(j * _LANES, _LANES)
                plsc.addupdate(o_v.at[sl], t_v[sl])
                return 0

            lax.fori_loop(0, chunk_elems // _LANES, add_body, 0)
            pltpu.sync_copy(o_v, out_hbm.at[pl.ds(xoff, chunk_elems)])
            return 0

        lax.fori_loop(0, nch, chunk_body, 0)

    return k(x2, t2)


def kernel(x, embeddings_table):
    B, S, D = x.shape
    x2 = x.reshape(B * S * D)
    t2 = embeddings_table.reshape(-1)
    out = _sc_add(x2, t2, B, S)
    return out.reshape(B, S, D)


# speedup vs baseline: 1.5074x; 1.5074x over previous
import functools

import jax
import jax.numpy as jnp
from jax import lax
from jax.experimental import pallas as pl
from jax.experimental.pallas import tpu as pltpu
from jax.experimental.pallas import tpu_sc as plsc

_D = 1024
_LANES = 16
_CH = 16


def _sc_add(x2, t2, B, S):
    R = B * S
    info = plsc.get_sparse_core_info()
    NC, NS = info.num_cores, info.num_subcores
    NW = NC * NS
    rows_per_w = R // NW
    nch = rows_per_w // _CH
    CE = _CH * _D

    mesh = plsc.VectorSubcoreMesh(core_axis_name="c", subcore_axis_name="s")

    @functools.partial(
        pl.kernel,
        mesh=mesh,
        out_type=jax.ShapeDtypeStruct((R * _D,), jnp.float32),
        scratch_types=[
            pltpu.VMEM((CE,), jnp.float32),
            pltpu.VMEM((CE,), jnp.float32),
            pltpu.VMEM((CE,), jnp.float32),
            pltpu.VMEM((CE,), jnp.float32),
            pltpu.SemaphoreType.DMA,
            pltpu.SemaphoreType.DMA,
            pltpu.SemaphoreType.DMA,
            pltpu.SemaphoreType.DMA,
            pltpu.SemaphoreType.DMA,
            pltpu.SemaphoreType.DMA,
        ],
    )
    def k(x_hbm, t_hbm, out_hbm, o_v0, o_v1, t_v0, t_v1,
          sx0, sx1, st0, st1, so0, so1):
        wid = lax.axis_index("s") * NC + lax.axis_index("c")
        row0 = wid * rows_per_w
        s0 = row0 % S
        o_bufs, t_bufs = (o_v0, o_v1), (t_v0, t_v1)
        sx, st, so = (sx0, sx1), (st0, st1), (so0, so1)

        def in_copies(i, b):
            xoff = (row0 + i * _CH) * _D
            toff = (s0 + i * _CH) * _D
            return (
                pltpu.make_async_copy(x_hbm.at[pl.ds(xoff, CE)], o_bufs[b], sx[b]),
                pltpu.make_async_copy(t_hbm.at[pl.ds(toff, CE)], t_bufs[b], st[b]),
            )

        def out_copy(i, b):
            xoff = (row0 + i * _CH) * _D
            return pltpu.make_async_copy(o_bufs[b], out_hbm.at[pl.ds(xoff, CE)], so[b])

        cx, ct = in_copies(0, 0)
        cx.start()
        ct.start()
        for i in range(nch):
            b = i % 2
            if i + 1 < nch:
                if i >= 1:
                    out_copy(i - 1, 1 - b).wait()
                ncx, nct = in_copies(i + 1, 1 - b)
                ncx.start()
                nct.start()
            cx, ct = in_copies(i, b)
            cx.wait()
            ct.wait()
            o_v, t_v = o_bufs[b], t_bufs[b]

            @plsc.parallel_loop(0, CE, _LANES, unroll=8)
            def add_body(off):
                plsc.addupdate(o_v.at[pl.ds(off, _LANES)], t_v[pl.ds(off, _LANES)])

            out_copy(i, b).start()
        out_copy(nch - 1, (nch - 1) % 2).wait()

    return k(x2, t2)


def kernel(x, embeddings_table):
    B, S, D = x.shape
    x2 = x.reshape(B * S * D)
    t2 = embeddings_table.reshape(-1)
    out = _sc_add(x2, t2, B, S)
    return out.reshape(B, S, D)


# speedup vs baseline: 3.2363x; 2.1469x over previous
import functools

import jax
import jax.numpy as jnp
from jax import lax
from jax.experimental import pallas as pl
from jax.experimental.pallas import tpu as pltpu
from jax.experimental.pallas import tpu_sc as plsc

_LANES = 16
_CH = 16


def _sc_add(x, t, B, S, D):
    R = B * S
    info = plsc.get_sparse_core_info()
    NC, NS = info.num_cores, info.num_subcores
    NW = NC * NS
    rows_per_w = R // NW
    nch = rows_per_w // _CH

    mesh = plsc.VectorSubcoreMesh(core_axis_name="c", subcore_axis_name="s")

    @functools.partial(
        pl.kernel,
        mesh=mesh,
        out_type=jax.ShapeDtypeStruct((B, S, D), jnp.float32),
        scratch_types=[
            pltpu.VMEM((_CH, D), jnp.float32),
            pltpu.VMEM((_CH, D), jnp.float32),
            pltpu.VMEM((_CH, D), jnp.float32),
            pltpu.VMEM((_CH, D), jnp.float32),
            pltpu.SemaphoreType.DMA,
            pltpu.SemaphoreType.DMA,
            pltpu.SemaphoreType.DMA,
            pltpu.SemaphoreType.DMA,
            pltpu.SemaphoreType.DMA,
            pltpu.SemaphoreType.DMA,
        ],
    )
    def k(x_hbm, t_hbm, out_hbm, o_v0, o_v1, t_v0, t_v1,
          sx0, sx1, st0, st1, so0, so1):
        wid = lax.axis_index("s") * NC + lax.axis_index("c")
        row0 = wid * rows_per_w
        bat = row0 // S
        s0 = row0 % S
        o_bufs, t_bufs = (o_v0, o_v1), (t_v0, t_v1)
        sx, st, so = (sx0, sx1), (st0, st1), (so0, so1)

        def in_copies(i, b):
            s_lo = s0 + i * _CH
            return (
                pltpu.make_async_copy(
                    x_hbm.at[bat, pl.ds(s_lo, _CH), :], o_bufs[b], sx[b]),
                pltpu.make_async_copy(
                    t_hbm.at[pl.ds(s_lo, _CH), :], t_bufs[b], st[b]),
            )

        def out_copy(i, b):
            s_lo = s0 + i * _CH
            return pltpu.make_async_copy(
                o_bufs[b], out_hbm.at[bat, pl.ds(s_lo, _CH), :], so[b])

        cx, ct = in_copies(0, 0)
        cx.start()
        ct.start()
        for i in range(nch):
            b = i % 2
            if i + 1 < nch:
                if i >= 1:
                    out_copy(i - 1, 1 - b).wait()
                ncx, nct = in_copies(i + 1, 1 - b)
                ncx.start()
                nct.start()
            cx, ct = in_copies(i, b)
            cx.wait()
            ct.wait()
            o_v, t_v = o_bufs[b], t_bufs[b]

            @plsc.parallel_loop(0, _CH * D, _LANES, unroll=8)
            def add_body(off):
                r = off // D
                c = off % D
                plsc.addupdate(o_v.at[r, pl.ds(c, _LANES)],
                               t_v[r, pl.ds(c, _LANES)])

            out_copy(i, b).start()
        out_copy(nch - 1, (nch - 1) % 2).wait()

    return k(x, t)


def kernel(x, embeddings_table):
    B, S, D = x.shape
    return _sc_add(x, embeddings_table, B, S, D)


# speedup vs baseline: 3.6978x; 1.1426x over previous
import functools

import jax
import jax.numpy as jnp
from jax import lax
from jax.experimental import pallas as pl
from jax.experimental.pallas import tpu as pltpu
from jax.experimental.pallas import tpu_sc as plsc

_LANES = 16
_CH = 16


def _sc_add(x, t, B, S, D):
    info = plsc.get_sparse_core_info()
    NC, NS = info.num_cores, info.num_subcores
    NW = NC * NS
    s_per_w = S // NW
    nch = s_per_w // _CH
    nsteps = nch * B

    mesh = plsc.VectorSubcoreMesh(core_axis_name="c", subcore_axis_name="s")

    @functools.partial(
        pl.kernel,
        mesh=mesh,
        out_type=jax.ShapeDtypeStruct((B, S, D), jnp.float32),
        scratch_types=[
            pltpu.VMEM((_CH, D), jnp.float32),
            pltpu.VMEM((_CH, D), jnp.float32),
            pltpu.VMEM((_CH, D), jnp.float32),
            pltpu.VMEM((_CH, D), jnp.float32),
            pltpu.SemaphoreType.DMA,
            pltpu.SemaphoreType.DMA,
            pltpu.SemaphoreType.DMA,
            pltpu.SemaphoreType.DMA,
            pltpu.SemaphoreType.DMA,
            pltpu.SemaphoreType.DMA,
        ],
    )
    def k(x_hbm, t_hbm, out_hbm, o_v0, o_v1, t_v0, t_v1,
          sx0, sx1, st0, st1, so0, so1):
        wid = lax.axis_index("s") * NC + lax.axis_index("c")
        s0 = wid * s_per_w
        o_bufs, t_bufs = (o_v0, o_v1), (t_v0, t_v1)
        sx, st, so = (sx0, sx1), (st0, st1), (so0, so1)

        def x_copy(step, b):
            i, bat = step // B, step % B
            return pltpu.make_async_copy(
                x_hbm.at[bat, pl.ds(s0 + i * _CH, _CH), :], o_bufs[b], sx[b])

        def t_copy(i, b):
            return pltpu.make_async_copy(
                t_hbm.at[pl.ds(s0 + i * _CH, _CH), :], t_bufs[b], st[b])

        def out_copy(step, b):
            i, bat = step // B, step % B
            return pltpu.make_async_copy(
                o_bufs[b], out_hbm.at[bat, pl.ds(s0 + i * _CH, _CH), :], so[b])

        x_copy(0, 0).start()
        t_copy(0, 0).start()
        for step in range(nsteps):
            b = step % 2
            i = step // B
            if step + 1 < nsteps:
                if step >= 1:
                    out_copy(step - 1, 1 - b).wait()
                x_copy(step + 1, 1 - b).start()
            if step % B == 0 and i + 1 < nch:
                t_copy(i + 1, (i + 1) % 2).start()
            x_copy(step, b).wait()
            if step % B == 0:
                t_copy(i, i % 2).wait()
            o_v, t_v = o_bufs[b], t_bufs[i % 2]

            @plsc.parallel_loop(0, _CH * D, _LANES, unroll=8)
            def add_body(off):
                r = off // D
                c = off % D
                plsc.addupdate(o_v.at[r, pl.ds(c, _LANES)],
                               t_v[r, pl.ds(c, _LANES)])

            out_copy(step, b).start()
        out_copy(nsteps - 1, (nsteps - 1) % 2).wait()

    return k(x, t)


def kernel(x, embeddings_table):
    B, S, D = x.shape
    return _sc_add(x, embeddings_table, B, S, D)
